# baseline (speedup 1.0000x reference)
import functools

import jax
import jax.numpy as jnp
from jax import lax
from jax.experimental import pallas as pl
from jax.experimental.pallas import tpu as pltpu
from jax.experimental.pallas import tpu_sc as plsc

_NC = 2
_NS = 16
_NW = _NC * _NS
_CH = 128


def _swish(v):
    return v * jax.nn.sigmoid(v)


def _edge_stage(x, rbf, W_kj, b_kj, W_rbf, be):
    E, H = x.shape
    NR = rbf.shape[1]

    def body(x_r, rbf_r, wkj_r, bkj_r, wrbf_r, o_r):
        xk = _swish(
            jnp.dot(x_r[...], wkj_r[...], preferred_element_type=jnp.float32)
            + bkj_r[...]
        )
        rb = jnp.dot(rbf_r[...], wrbf_r[...], preferred_element_type=jnp.float32)
        o_r[...] = xk * rb

    return pl.pallas_call(
        body,
        grid=(E // be,),
        in_specs=[
            pl.BlockSpec((be, H), lambda i: (i, 0)),
            pl.BlockSpec((be, NR), lambda i: (i, 0)),
            pl.BlockSpec((H, H), lambda i: (0, 0)),
            pl.BlockSpec((1, H), lambda i: (0, 0)),
            pl.BlockSpec((NR, H), lambda i: (0, 0)),
        ],
        out_specs=pl.BlockSpec((be, H), lambda i: (i, 0)),
        out_shape=jax.ShapeDtypeStruct((E, H), jnp.float32),
        compiler_params=pltpu.CompilerParams(
            dimension_semantics=("parallel",)
        ),
    )(x, rbf, W_kj, b_kj.reshape(1, H), W_rbf)


def _sc_gather(table, idx):
    E, H = table.shape
    Tp = idx.shape[0]
    per_w = Tp // _NW
    n_ch = per_w // _CH
    idx3 = idx.reshape(_NW, n_ch, _CH)
    mesh = plsc.VectorSubcoreMesh(core_axis_name="c", subcore_axis_name="s")

    @functools.partial(
        pl.kernel,
        out_type=jax.ShapeDtypeStruct((Tp, H), jnp.float32),
        mesh=mesh,
        scratch_types=[
            pltpu.VMEM((n_ch, _CH), jnp.int32),
            pltpu.VMEM((_CH, H), jnp.float32),
            pltpu.SemaphoreType.DMA,
        ],
    )
    def k(table_hbm, idx_hbm, out_hbm, idx_v, rows_v, sem):
        wid = lax.axis_index("s") * _NC + lax.axis_index("c")
        base = wid * per_w
        pltpu.sync_copy(idx_hbm.at[wid], idx_v)

        def body(i, carry):
            pltpu.async_copy(table_hbm.at[idx_v.at[i]], rows_v, sem).wait()
            pltpu.sync_copy(rows_v, out_hbm.at[pl.ds(base + i * _CH, _CH)])
            return carry

        lax.fori_loop(0, n_ch, body, 0, unroll=False)

    return k(table, idx3)


def _triplet_stage(g, sbf, W_sbf, Wt, bt):
    Tp, H = g.shape
    NSR = sbf.shape[1]
    NB = Wt.shape[0]
    P = H // 8

    def body(g_r, sbf_r, wsbf_r, wt_r, o_r):
        sbf2 = jnp.dot(sbf_r[...], wsbf_r[...], preferred_element_type=jnp.float32)
        gv = g_r[...]
        acc = jnp.zeros((bt, H), jnp.float32)
        for j in range(NB):
            acc = acc + jnp.dot(
                gv, wt_r[j], preferred_element_type=jnp.float32
            ) * sbf2[:, j : j + 1]
        o_r[...] = acc.reshape(bt, P, 8).transpose(1, 0, 2)

    return pl.pallas_call(
        body,
        grid=(Tp // bt,),
        in_specs=[
            pl.BlockSpec((bt, H), lambda i: (i, 0)),
            pl.BlockSpec((bt, NSR), lambda i: (i, 0)),
            pl.BlockSpec((NSR, NB), lambda i: (0, 0)),
            pl.BlockSpec((NB, H, H), lambda i: (0, 0, 0)),
        ],
        out_specs=pl.BlockSpec((P, bt, 8), lambda i: (0, i, 0)),
        out_shape=jax.ShapeDtypeStruct((P, Tp, 8), jnp.float32),
        compiler_params=pltpu.CompilerParams(
            dimension_semantics=("parallel",)
        ),
    )(g, sbf, W_sbf, Wt)


def _sc_scatter(y_t, idx, E):
    P, Tp, F = y_t.shape
    per_tile = Tp // _NS
    n_ch = per_tile // _CH
    e_per_tile = E // _NS
    ECH = 2000
    n_ech = e_per_tile // ECH
    half = P // _NC
    idx3 = idx.reshape(_NS, n_ch, _CH)
    zeros = jnp.zeros((ECH, F), jnp.float32)
    mesh = plsc.VectorSubcoreMesh(core_axis_name="c", subcore_axis_name="s")

    @functools.partial(
        pl.kernel,
        out_type=jax.ShapeDtypeStruct((P, E, F), jnp.float32),
        mesh=mesh,
        scratch_types=[
            pltpu.VMEM((n_ch, _CH), jnp.int32),
            pltpu.VMEM((_CH, F), jnp.float32),
            pltpu.VMEM((ECH, F), jnp.float32),
            pltpu.VMEM((ECH, F), jnp.float32),
            pltpu.VMEM_SHARED((E, F), jnp.float32),
            pltpu.SemaphoreType.DMA,
        ],
        compiler_params=pltpu.CompilerParams(use_tc_tiling_on_sc=False),
    )
    def k(y_hbm, idx_hbm, z_hbm, out_hbm, idx_v, data_v, zbuf, fbuf, acc, sem):
        cid = lax.axis_index("c")
        sid = lax.axis_index("s")
        pltpu.sync_copy(idx_hbm.at[sid], idx_v)
        pltpu.sync_copy(z_hbm, zbuf)
        row0 = sid * per_tile
        e0 = sid * e_per_tile
        for p in range(half):
            pg = cid * half + p
            for q in range(n_ech):
                pltpu.sync_copy(zbuf, acc.at[pl.ds(e0 + q * ECH, ECH)])
            plsc.subcore_barrier()

            def body(i, carry):
                pltpu.sync_copy(
                    y_hbm.at[pg, pl.ds(row0 + i * _CH, _CH)], data_v
                )
                pltpu.sync_copy(data_v, acc.at[idx_v.at[i]], add=True)
                return carry

            lax.fori_loop(0, n_ch, body, 0, unroll=False)
            plsc.subcore_barrier()
            for q in range(n_ech):
                pltpu.sync_copy(acc.at[pl.ds(e0 + q * ECH, ECH)], fbuf)
                pltpu.sync_copy(fbuf, out_hbm.at[pg, pl.ds(e0 + q * ECH, ECH)])
            plsc.subcore_barrier()

    return k(y_t, idx3, zeros)


def _final_stage(x, scat_t, W_ji, b_ji, rb_w1, rb_b1, rb_w2, rb_b2,
                 W_lin, b_lin, ra1_w1, ra1_b1, ra1_w2, ra1_b2,
                 ra2_w1, ra2_b1, ra2_w2, ra2_b2, be):
    E, H = x.shape
    P = scat_t.shape[0]

    def body(x_r, sc_r, wji_r, bji_r, w1_r, B1_r, w2_r, B2_r,
             wl_r, bl_r, a1_r, ab1_r, a2_r, ab2_r, a3_r, ab3_r, a4_r, ab4_r,
             o_r):
        xv = x_r[...]
        scat = sc_r[...].transpose(1, 0, 2).reshape(be, H)
        h = _swish(jnp.dot(xv, wji_r[...], preferred_element_type=jnp.float32)
                   + bji_r[...]) + scat
        t = _swish(jnp.dot(h, w1_r[...], preferred_element_type=jnp.float32)
                   + B1_r[...])
        h = h + _swish(jnp.dot(t, w2_r[...], preferred_element_type=jnp.float32)
                       + B2_r[...])
        h = _swish(jnp.dot(h, wl_r[...], preferred_element_type=jnp.float32)
                   + bl_r[...]) + xv
        t = _swish(jnp.dot(h, a1_r[...], preferred_element_type=jnp.float32)
                   + ab1_r[...])
        h = h + _swish(jnp.dot(t, a2_r[...], preferred_element_type=jnp.float32)
                       + ab2_r[...])
        t = _swish(jnp.dot(h, a3_r[...], preferred_element_type=jnp.float32)
                   + ab3_r[...])
        h = h + _swish(jnp.dot(t, a4_r[...], preferred_element_type=jnp.float32)
                       + ab4_r[...])
        o_r[...] = h

    wfull = pl.BlockSpec((H, H), lambda i: (0, 0))
    bfull = pl.BlockSpec((1, H), lambda i: (0, 0))
    return pl.pallas_call(
        body,
        grid=(E // be,),
        in_specs=[
            pl.BlockSpec((be, H), lambda i: (i, 0)),
            pl.BlockSpec((P, be, 8), lambda i: (0, i, 0)),
            wfull, bfull, wfull, bfull, wfull, bfull,
            wfull, bfull, wfull, bfull, wfull, bfull,
            wfull, bfull, wfull, bfull,
        ],
        out_specs=pl.BlockSpec((be, H), lambda i: (i, 0)),
        out_shape=jax.ShapeDtypeStruct((E, H), jnp.float32),
        compiler_params=pltpu.CompilerParams(
            dimension_semantics=("parallel",)
        ),
    )(x, scat_t, W_ji, b_ji.reshape(1, H), rb_w1, rb_b1.reshape(1, H),
      rb_w2, rb_b2.reshape(1, H), W_lin, b_lin.reshape(1, H),
      ra1_w1, ra1_b1.reshape(1, H), ra1_w2, ra1_b2.reshape(1, H),
      ra2_w1, ra2_b1.reshape(1, H), ra2_w2, ra2_b2.reshape(1, H))


def kernel(x, rbf, sbf, idx_kj, idx_ji, W_rbf, W_sbf, W_kj, b_kj, W_ji, b_ji,
           W_bil, rb_w1, rb_b1, rb_w2, rb_b2, W_lin, b_lin,
           ra1_w1, ra1_b1, ra1_w2, ra1_b2, ra2_w1, ra2_b1, ra2_w2, ra2_b2):
    E, H = x.shape
    T = idx_kj.shape[0]
    ikj = idx_kj.astype(jnp.int32)
    iji = idx_ji.astype(jnp.int32)

    gran = _NW * _CH
    Tp = ((T + gran - 1) // gran) * gran
    pad = Tp - T
    if pad:
        fill = jnp.arange(pad, dtype=jnp.int32) % E
        ikj = jnp.concatenate([ikj, fill])
        iji = jnp.concatenate([iji, fill])
        sbf = jnp.pad(sbf, ((0, pad), (0, 0)))

    Wt = jnp.transpose(W_bil, (1, 2, 0))

    x_kj0 = _edge_stage(x, rbf, W_kj, b_kj, W_rbf, be=2000)
    g = _sc_gather(x_kj0, ikj)
    y_t = _triplet_stage(g, sbf, W_sbf, Wt, bt=2048)
    scat_t = _sc_scatter(y_t, iji, E)
    return _final_stage(x, scat_t, W_ji, b_ji, rb_w1, rb_b1, rb_w2, rb_b2,
                        W_lin, b_lin, ra1_w1, ra1_b1, ra1_w2, ra1_b2,
                        ra2_w1, ra2_b1, ra2_w2, ra2_b2, be=2000)


# speedup vs baseline: 2.9131x; 2.9131x over previous
import functools

import jax
import jax.numpy as jnp
from jax import lax
from jax.experimental import pallas as pl
from jax.experimental.pallas import tpu as pltpu
from jax.experimental.pallas import tpu_sc as plsc

_NC = 2
_NS = 16
_NW = _NC * _NS
_CH = 128


def _swish(v):
    return v * jax.nn.sigmoid(v)


def _edge_stage(x, rbf, W_kj, b_kj, W_rbf, be):
    E, H = x.shape
    NR = rbf.shape[1]

    def body(x_r, rbf_r, wkj_r, bkj_r, wrbf_r, o_r):
        xk = _swish(
            jnp.dot(x_r[...], wkj_r[...], preferred_element_type=jnp.float32)
            + bkj_r[...]
        )
        rb = jnp.dot(rbf_r[...], wrbf_r[...], preferred_element_type=jnp.float32)
        o_r[...] = xk * rb

    return pl.pallas_call(
        body,
        grid=(E // be,),
        in_specs=[
            pl.BlockSpec((be, H), lambda i: (i, 0)),
            pl.BlockSpec((be, NR), lambda i: (i, 0)),
            pl.BlockSpec((H, H), lambda i: (0, 0)),
            pl.BlockSpec((1, H), lambda i: (0, 0)),
            pl.BlockSpec((NR, H), lambda i: (0, 0)),
        ],
        out_specs=pl.BlockSpec((be, H), lambda i: (i, 0)),
        out_shape=jax.ShapeDtypeStruct((E, H), jnp.float32),
        compiler_params=pltpu.CompilerParams(
            dimension_semantics=("parallel",)
        ),
    )(x, rbf, W_kj, b_kj.reshape(1, H), W_rbf)


def _sc_gather(table, idx):
    E, H = table.shape
    Tp = idx.shape[0]
    per_w = Tp // _NW
    n_ch = per_w // _CH
    idx3 = idx.reshape(_NW, n_ch, _CH)
    mesh = plsc.VectorSubcoreMesh(core_axis_name="c", subcore_axis_name="s")

    @functools.partial(
        pl.kernel,
        out_type=jax.ShapeDtypeStruct((Tp, H), jnp.float32),
        mesh=mesh,
        scratch_types=[
            pltpu.VMEM((n_ch, _CH), jnp.int32),
            pltpu.VMEM((_CH, H), jnp.float32),
            pltpu.SemaphoreType.DMA,
        ],
    )
    def k(table_hbm, idx_hbm, out_hbm, idx_v, rows_v, sem):
        wid = lax.axis_index("s") * _NC + lax.axis_index("c")
        base = wid * per_w
        pltpu.sync_copy(idx_hbm.at[wid], idx_v)

        def body(i, carry):
            pltpu.async_copy(table_hbm.at[idx_v.at[i]], rows_v, sem).wait()
            pltpu.sync_copy(rows_v, out_hbm.at[pl.ds(base + i * _CH, _CH)])
            return carry

        lax.fori_loop(0, n_ch, body, 0, unroll=False)

    return k(table, idx3)


def _triplet_stage(g, sbf, W_sbf, Wbig, bt):
    Tp, H = g.shape
    NSR = sbf.shape[1]
    NB = Wbig.shape[1] // H

    def body(g_r, sbf_r, wsbf_r, wbig_r, o_r):
        sbf2 = jnp.dot(sbf_r[...], wsbf_r[...], preferred_element_type=jnp.float32)
        big = jnp.dot(g_r[...], wbig_r[...], preferred_element_type=jnp.float32)
        acc = jnp.zeros((bt, H), jnp.float32)
        for j in range(NB):
            acc = acc + big[:, j * H : (j + 1) * H] * sbf2[:, j : j + 1]
        o_r[...] = acc

    return pl.pallas_call(
        body,
        grid=(Tp // bt,),
        in_specs=[
            pl.BlockSpec((bt, H), lambda i: (i, 0)),
            pl.BlockSpec((bt, NSR), lambda i: (i, 0)),
            pl.BlockSpec((NSR, NB), lambda i: (0, 0)),
            pl.BlockSpec((H, NB * H), lambda i: (0, 0)),
        ],
        out_specs=pl.BlockSpec((bt, H), lambda i: (i, 0)),
        out_shape=jax.ShapeDtypeStruct((Tp, H), jnp.float32),
        compiler_params=pltpu.CompilerParams(
            dimension_semantics=("parallel",)
        ),
    )(g, sbf, W_sbf, Wbig)


def _sc_scatter(y, idx, E):
    Tp, H = y.shape
    F = 8
    P = H // F
    per_tile = Tp // _NS
    n_ch = per_tile // _CH
    e_per_tile = E // _NS
    ECH = 2000
    n_ech = e_per_tile // ECH
    half = P // _NC
    idx3 = idx.reshape(_NS, n_ch, _CH)
    zeros = jnp.zeros((ECH, F), jnp.float32)
    mesh = plsc.VectorSubcoreMesh(core_axis_name="c", subcore_axis_name="s")

    @functools.partial(
        pl.kernel,
        out_type=jax.ShapeDtypeStruct((E, H), jnp.float32),
        mesh=mesh,
        scratch_types=[
            pltpu.VMEM((n_ch, _CH), jnp.int32),
            pltpu.VMEM((_CH, F), jnp.float32),
            pltpu.VMEM((ECH, F), jnp.float32),
            pltpu.VMEM((ECH, F), jnp.float32),
            pltpu.VMEM_SHARED((E, F), jnp.float32),
            pltpu.SemaphoreType.DMA,
        ],
        compiler_params=pltpu.CompilerParams(use_tc_tiling_on_sc=False),
    )
    def k(y_hbm, idx_hbm, z_hbm, out_hbm, idx_v, data_v, zbuf, fbuf, acc, sem):
        cid = lax.axis_index("c")
        sid = lax.axis_index("s")
        pltpu.sync_copy(idx_hbm.at[sid], idx_v)
        pltpu.sync_copy(z_hbm, zbuf)
        row0 = sid * per_tile
        e0 = sid * e_per_tile
        for p in range(half):
            col0 = (cid * half + p) * F
            for q in range(n_ech):
                pltpu.sync_copy(zbuf, acc.at[pl.ds(e0 + q * ECH, ECH)])
            plsc.subcore_barrier()

            def body(i, carry):
                pltpu.sync_copy(
                    y_hbm.at[pl.ds(row0 + i * _CH, _CH), pl.ds(col0, F)],
                    data_v,
                )
                pltpu.sync_copy(data_v, acc.at[idx_v.at[i]], add=True)
                return carry

            lax.fori_loop(0, n_ch, body, 0, unroll=False)
            plsc.subcore_barrier()
            for q in range(n_ech):
                pltpu.sync_copy(acc.at[pl.ds(e0 + q * ECH, ECH)], fbuf)
                pltpu.sync_copy(
                    fbuf,
                    out_hbm.at[pl.ds(e0 + q * ECH, ECH), pl.ds(col0, F)],
                )
            plsc.subcore_barrier()

    return k(y, idx3, zeros)


def _final_stage(x, scat, W_ji, b_ji, rb_w1, rb_b1, rb_w2, rb_b2,
                 W_lin, b_lin, ra1_w1, ra1_b1, ra1_w2, ra1_b2,
                 ra2_w1, ra2_b1, ra2_w2, ra2_b2, be):
    E, H = x.shape

    def body(x_r, sc_r, wji_r, bji_r, w1_r, B1_r, w2_r, B2_r,
             wl_r, bl_r, a1_r, ab1_r, a2_r, ab2_r, a3_r, ab3_r, a4_r, ab4_r,
             o_r):
        xv = x_r[...]
        h = _swish(jnp.dot(xv, wji_r[...], preferred_element_type=jnp.float32)
                   + bji_r[...]) + sc_r[...]
        t = _swish(jnp.dot(h, w1_r[...], preferred_element_type=jnp.float32)
                   + B1_r[...])
        h = h + _swish(jnp.dot(t, w2_r[...], preferred_element_type=jnp.float32)
                       + B2_r[...])
        h = _swish(jnp.dot(h, wl_r[...], preferred_element_type=jnp.float32)
                   + bl_r[...]) + xv
        t = _swish(jnp.dot(h, a1_r[...], preferred_element_type=jnp.float32)
                   + ab1_r[...])
        h = h + _swish(jnp.dot(t, a2_r[...], preferred_element_type=jnp.float32)
                       + ab2_r[...])
        t = _swish(jnp.dot(h, a3_r[...], preferred_element_type=jnp.float32)
                   + ab3_r[...])
        h = h + _swish(jnp.dot(t, a4_r[...], preferred_element_type=jnp.float32)
                       + ab4_r[...])
        o_r[...] = h

    wfull = pl.BlockSpec((H, H), lambda i: (0, 0))
    bfull = pl.BlockSpec((1, H), lambda i: (0, 0))
    return pl.pallas_call(
        body,
        grid=(E // be,),
        in_specs=[
            pl.BlockSpec((be, H), lambda i: (i, 0)),
            pl.BlockSpec((be, H), lambda i: (i, 0)),
            wfull, bfull, wfull, bfull, wfull, bfull,
            wfull, bfull, wfull, bfull, wfull, bfull,
            wfull, bfull, wfull, bfull,
        ],
        out_specs=pl.BlockSpec((be, H), lambda i: (i, 0)),
        out_shape=jax.ShapeDtypeStruct((E, H), jnp.float32),
        compiler_params=pltpu.CompilerParams(
            dimension_semantics=("parallel",)
        ),
    )(x, scat, W_ji, b_ji.reshape(1, H), rb_w1, rb_b1.reshape(1, H),
      rb_w2, rb_b2.reshape(1, H), W_lin, b_lin.reshape(1, H),
      ra1_w1, ra1_b1.reshape(1, H), ra1_w2, ra1_b2.reshape(1, H),
      ra2_w1, ra2_b1.reshape(1, H), ra2_w2, ra2_b2.reshape(1, H))


def kernel(x, rbf, sbf, idx_kj, idx_ji, W_rbf, W_sbf, W_kj, b_kj, W_ji, b_ji,
           W_bil, rb_w1, rb_b1, rb_w2, rb_b2, W_lin, b_lin,
           ra1_w1, ra1_b1, ra1_w2, ra1_b2, ra2_w1, ra2_b1, ra2_w2, ra2_b2):
    E, H = x.shape
    T = idx_kj.shape[0]
    ikj = idx_kj.astype(jnp.int32)
    iji = idx_ji.astype(jnp.int32)

    gran = _NW * _CH
    Tp = ((T + gran - 1) // gran) * gran
    pad = Tp - T
    if pad:
        fill = jnp.arange(pad, dtype=jnp.int32) % E
        ikj = jnp.concatenate([ikj, fill])
        iji = jnp.concatenate([iji, fill])
        sbf = jnp.pad(sbf, ((0, pad), (0, 0)))

    NB = W_bil.shape[1]
    Wbig = jnp.transpose(W_bil, (2, 1, 0)).reshape(H, NB * H)

    x_kj0 = _edge_stage(x, rbf, W_kj, b_kj, W_rbf, be=2000)
    g = _sc_gather(x_kj0, ikj)
    y = _triplet_stage(g, sbf, W_sbf, Wbig, bt=2048)
    scat = _sc_scatter(y, iji, E)
    return _final_stage(x, scat, W_ji, b_ji, rb_w1, rb_b1, rb_w2, rb_b2,
                        W_lin, b_lin, ra1_w1, ra1_b1, ra1_w2, ra1_b2,
                        ra2_w1, ra2_b1, ra2_w2, ra2_b2, be=2000)


# speedup vs baseline: 4.0333x; 1.3845x over previous
import functools

import jax
import jax.numpy as jnp
from jax import lax
from jax.experimental import pallas as pl
from jax.experimental.pallas import tpu as pltpu
from jax.experimental.pallas import tpu_sc as plsc

_NC = 2
_NS = 16
_NW = _NC * _NS
_CH = 128


def _swish(v):
    return v * jax.nn.sigmoid(v)


def _edge_stage(x, rbf, W_kj, b_kj, W_rbf, be):
    E, H = x.shape
    NR = rbf.shape[1]

    def body(x_r, rbf_r, wkj_r, bkj_r, wrbf_r, o_r):
        xk = _swish(
            jnp.dot(x_r[...], wkj_r[...], preferred_element_type=jnp.float32)
            + bkj_r[...]
        )
        rb = jnp.dot(rbf_r[...], wrbf_r[...], preferred_element_type=jnp.float32)
        o_r[...] = xk * rb

    return pl.pallas_call(
        body,
        grid=(E // be,),
        in_specs=[
            pl.BlockSpec((be, H), lambda i: (i, 0)),
            pl.BlockSpec((be, NR), lambda i: (i, 0)),
            pl.BlockSpec((H, H), lambda i: (0, 0)),
            pl.BlockSpec((1, H), lambda i: (0, 0)),
            pl.BlockSpec((NR, H), lambda i: (0, 0)),
        ],
        out_specs=pl.BlockSpec((be, H), lambda i: (i, 0)),
        out_shape=jax.ShapeDtypeStruct((E, H), jnp.float32),
        compiler_params=pltpu.CompilerParams(
            dimension_semantics=("parallel",)
        ),
    )(x, rbf, W_kj, b_kj.reshape(1, H), W_rbf)


def _sc_gather(table, idx):
    E, H = table.shape
    Tp = idx.shape[0]
    per_w = Tp // _NW
    n_ch = per_w // _CH
    idx3 = idx.reshape(_NW, n_ch, _CH)
    mesh = plsc.VectorSubcoreMesh(core_axis_name="c", subcore_axis_name="s")

    @functools.partial(
        pl.kernel,
        out_type=jax.ShapeDtypeStruct((Tp, H), jnp.float32),
        mesh=mesh,
        scratch_types=[
            pltpu.VMEM((n_ch, _CH), jnp.int32),
            pltpu.VMEM((_CH, H), jnp.float32),
            pltpu.SemaphoreType.DMA,
        ],
    )
    def k(table_hbm, idx_hbm, out_hbm, idx_v, rows_v, sem):
        wid = lax.axis_index("s") * _NC + lax.axis_index("c")
        base = wid * per_w
        pltpu.sync_copy(idx_hbm.at[wid], idx_v)

        def body(i, carry):
            pltpu.async_copy(table_hbm.at[idx_v.at[i]], rows_v, sem).wait()
            pltpu.sync_copy(rows_v, out_hbm.at[pl.ds(base + i * _CH, _CH)])
            return carry

        lax.fori_loop(0, n_ch, body, 0, unroll=False)

    return k(table, idx3)


def _triplet_stage(g, sbf, W_sbf, Wbig, bt):
    Tp, H = g.shape
    NSR = sbf.shape[1]
    NB = Wbig.shape[1] // H

    def body(g_r, sbf_r, wsbf_r, wbig_r, o_r):
        sbf2 = jnp.dot(sbf_r[...], wsbf_r[...], preferred_element_type=jnp.float32)
        big = jnp.dot(g_r[...], wbig_r[...], preferred_element_type=jnp.float32)
        acc = jnp.zeros((bt, H), jnp.float32)
        for j in range(NB):
            acc = acc + big[:, j * H : (j + 1) * H] * sbf2[:, j : j + 1]
        o_r[...] = acc

    return pl.pallas_call(
        body,
        grid=(Tp // bt,),
        in_specs=[
            pl.BlockSpec((bt, H), lambda i: (i, 0)),
            pl.BlockSpec((bt, NSR), lambda i: (i, 0)),
            pl.BlockSpec((NSR, NB), lambda i: (0, 0)),
            pl.BlockSpec((H, NB * H), lambda i: (0, 0)),
        ],
        out_specs=pl.BlockSpec((bt, H), lambda i: (i, 0)),
        out_shape=jax.ShapeDtypeStruct((Tp, H), jnp.float32),
        compiler_params=pltpu.CompilerParams(
            dimension_semantics=("parallel",)
        ),
    )(g, sbf, W_sbf, Wbig)


def _sc_scatter(y, idx, E):
    Tp, H = y.shape
    F = 8
    P = H // F
    per_tile = Tp // _NS
    n_ch = per_tile // _CH
    e_per_tile = E // _NS
    ECH = 2000
    n_ech = e_per_tile // ECH
    half = P // _NC
    idx3 = idx.reshape(_NS, n_ch, _CH)
    zeros = jnp.zeros((ECH, F), jnp.float32)
    mesh = plsc.VectorSubcoreMesh(core_axis_name="c", subcore_axis_name="s")

    NBUF = 4

    @functools.partial(
        pl.kernel,
        out_type=jax.ShapeDtypeStruct((E, H), jnp.float32),
        mesh=mesh,
        scratch_types=[
            pltpu.VMEM((n_ch, _CH), jnp.int32),
            [pltpu.VMEM((_CH, F), jnp.float32) for _ in range(NBUF)],
            pltpu.VMEM((ECH, F), jnp.float32),
            pltpu.VMEM((ECH, F), jnp.float32),
            pltpu.VMEM_SHARED((E, F), jnp.float32),
            [pltpu.SemaphoreType.DMA for _ in range(NBUF)],
            pltpu.SemaphoreType.DMA,
        ],
        compiler_params=pltpu.CompilerParams(use_tc_tiling_on_sc=False),
    )
    def k(y_hbm, idx_hbm, z_hbm, out_hbm, idx_v, data_v, zbuf, fbuf, acc,
          semr, sem):
        cid = lax.axis_index("c")
        sid = lax.axis_index("s")
        pltpu.sync_copy(idx_hbm.at[sid], idx_v)
        pltpu.sync_copy(z_hbm, zbuf)
        row0 = sid * per_tile
        e0 = sid * e_per_tile

        def rd_src(col0, i):
            return y_hbm.at[pl.ds(row0 + i * _CH, _CH), pl.ds(col0, F)]

        for p in range(half):
            col0 = (cid * half + p) * F
            for q in range(n_ech):
                pltpu.sync_copy(zbuf, acc.at[pl.ds(e0 + q * ECH, ECH)])
            plsc.subcore_barrier()

            for b in range(NBUF):
                pltpu.async_copy(rd_src(col0, b), data_v[b], semr[b])

            def body(i2, carry):
                for b in range(NBUF):
                    i = i2 * NBUF + b
                    pltpu.make_async_copy(
                        rd_src(col0, i), data_v[b], semr[b]
                    ).wait()
                    pltpu.sync_copy(data_v[b], acc.at[idx_v.at[i]], add=True)

                    @pl.when(i + NBUF < n_ch)
                    def _():
                        pltpu.async_copy(
                            rd_src(col0, i + NBUF), data_v[b], semr[b]
                        )
                return carry

            lax.fori_loop(0, n_ch // NBUF, body, 0, unroll=False)
            plsc.subcore_barrier()
            for q in range(n_ech):
                pltpu.sync_copy(acc.at[pl.ds(e0 + q * ECH, ECH)], fbuf)
                pltpu.sync_copy(
                    fbuf,
                    out_hbm.at[pl.ds(e0 + q * ECH, ECH), pl.ds(col0, F)],
                )
            plsc.subcore_barrier()

    return k(y, idx3, zeros)


def _final_stage(x, scat, W_ji, b_ji, rb_w1, rb_b1, rb_w2, rb_b2,
                 W_lin, b_lin, ra1_w1, ra1_b1, ra1_w2, ra1_b2,
                 ra2_w1, ra2_b1, ra2_w2, ra2_b2, be):
    E, H = x.shape

    def body(x_r, sc_r, wji_r, bji_r, w1_r, B1_r, w2_r, B2_r,
             wl_r, bl_r, a1_r, ab1_r, a2_r, ab2_r, a3_r, ab3_r, a4_r, ab4_r,
             o_r):
        xv = x_r[...]
        h = _swish(jnp.dot(xv, wji_r[...], preferred_element_type=jnp.float32)
                   + bji_r[...]) + sc_r[...]
        t = _swish(jnp.dot(h, w1_r[...], preferred_element_type=jnp.float32)
                   + B1_r[...])
        h = h + _swish(jnp.dot(t, w2_r[...], preferred_element_type=jnp.float32)
                       + B2_r[...])
        h = _swish(jnp.dot(h, wl_r[...], preferred_element_type=jnp.float32)
                   + bl_r[...]) + xv
        t = _swish(jnp.dot(h, a1_r[...], preferred_element_type=jnp.float32)
                   + ab1_r[...])
        h = h + _swish(jnp.dot(t, a2_r[...], preferred_element_type=jnp.float32)
                       + ab2_r[...])
        t = _swish(jnp.dot(h, a3_r[...], preferred_element_type=jnp.float32)
                   + ab3_r[...])
        h = h + _swish(jnp.dot(t, a4_r[...], preferred_element_type=jnp.float32)
                       + ab4_r[...])
        o_r[...] = h

    wfull = pl.BlockSpec((H, H), lambda i: (0, 0))
    bfull = pl.BlockSpec((1, H), lambda i: (0, 0))
    return pl.pallas_call(
        body,
        grid=(E // be,),
        in_specs=[
            pl.BlockSpec((be, H), lambda i: (i, 0)),
            pl.BlockSpec((be, H), lambda i: (i, 0)),
            wfull, bfull, wfull, bfull, wfull, bfull,
            wfull, bfull, wfull, bfull, wfull, bfull,
            wfull, bfull, wfull, bfull,
        ],
        out_specs=pl.BlockSpec((be, H), lambda i: (i, 0)),
        out_shape=jax.ShapeDtypeStruct((E, H), jnp.float32),
        compiler_params=pltpu.CompilerParams(
            dimension_semantics=("parallel",)
        ),
    )(x, scat, W_ji, b_ji.reshape(1, H), rb_w1, rb_b1.reshape(1, H),
      rb_w2, rb_b2.reshape(1, H), W_lin, b_lin.reshape(1, H),
      ra1_w1, ra1_b1.reshape(1, H), ra1_w2, ra1_b2.reshape(1, H),
      ra2_w1, ra2_b1.reshape(1, H), ra2_w2, ra2_b2.reshape(1, H))


def kernel(x, rbf, sbf, idx_kj, idx_ji, W_rbf, W_sbf, W_kj, b_kj, W_ji, b_ji,
           W_bil, rb_w1, rb_b1, rb_w2, rb_b2, W_lin, b_lin,
           ra1_w1, ra1_b1, ra1_w2, ra1_b2, ra2_w1, ra2_b1, ra2_w2, ra2_b2):
    E, H = x.shape
    T = idx_kj.shape[0]
    ikj = idx_kj.astype(jnp.int32)
    iji = idx_ji.astype(jnp.int32)

    gran = _NW * _CH
    Tp = ((T + gran - 1) // gran) * gran
    pad = Tp - T
    if pad:
        fill = jnp.arange(pad, dtype=jnp.int32) % E
        ikj = jnp.concatenate([ikj, fill])
        iji = jnp.concatenate([iji, fill])
        sbf = jnp.pad(sbf, ((0, pad), (0, 0)))

    NB = W_bil.shape[1]
    Wbig = jnp.transpose(W_bil, (2, 1, 0)).reshape(H, NB * H)

    x_kj0 = _edge_stage(x, rbf, W_kj, b_kj, W_rbf, be=2000)
    g = _sc_gather(x_kj0, ikj)
    y = _triplet_stage(g, sbf, W_sbf, Wbig, bt=2048)
    scat = _sc_scatter(y, iji, E)
    return _final_stage(x, scat, W_ji, b_ji, rb_w1, rb_b1, rb_w2, rb_b2,
                        W_lin, b_lin, ra1_w1, ra1_b1, ra1_w2, ra1_b2,
                        ra2_w1, ra2_b1, ra2_w2, ra2_b2, be=2000)


# speedup vs baseline: 4.0982x; 1.0161x over previous
import functools

import jax
import jax.numpy as jnp
from jax import lax
from jax.experimental import pallas as pl
from jax.experimental.pallas import tpu as pltpu
from jax.experimental.pallas import tpu_sc as plsc

_NC = 2
_NS = 16
_NW = _NC * _NS
_CH = 128


def _swish(v):
    return v * jax.nn.sigmoid(v)


def _edge_stage(x, rbf, W_kj, b_kj, W_rbf, be):
    E, H = x.shape
    NR = rbf.shape[1]

    def body(x_r, rbf_r, wkj_r, bkj_r, wrbf_r, o_r):
        xk = _swish(
            jnp.dot(x_r[...], wkj_r[...], preferred_element_type=jnp.float32)
            + bkj_r[...]
        )
        rb = jnp.dot(rbf_r[...], wrbf_r[...], preferred_element_type=jnp.float32)
        o_r[...] = xk * rb

    return pl.pallas_call(
        body,
        grid=(E // be,),
        in_specs=[
            pl.BlockSpec((be, H), lambda i: (i, 0)),
            pl.BlockSpec((be, NR), lambda i: (i, 0)),
            pl.BlockSpec((H, H), lambda i: (0, 0)),
            pl.BlockSpec((1, H), lambda i: (0, 0)),
            pl.BlockSpec((NR, H), lambda i: (0, 0)),
        ],
        out_specs=pl.BlockSpec((be, H), lambda i: (i, 0)),
        out_shape=jax.ShapeDtypeStruct((E, H), jnp.float32),
        compiler_params=pltpu.CompilerParams(
            dimension_semantics=("parallel",)
        ),
    )(x, rbf, W_kj, b_kj.reshape(1, H), W_rbf)


def _sc_gather(table, idx):
    E, H = table.shape
    Tp = idx.shape[0]
    per_w = Tp // _NW
    n_ch = per_w // _CH
    idx3 = idx.reshape(_NW, n_ch, _CH)
    mesh = plsc.VectorSubcoreMesh(core_axis_name="c", subcore_axis_name="s")

    NBUF = 4
    assert n_ch % NBUF == 0

    @functools.partial(
        pl.kernel,
        out_type=jax.ShapeDtypeStruct((Tp, H), jnp.float32),
        mesh=mesh,
        scratch_types=[
            pltpu.VMEM((n_ch, _CH), jnp.int32),
            [pltpu.VMEM((_CH, H), jnp.float32) for _ in range(NBUF)],
            [pltpu.SemaphoreType.DMA for _ in range(NBUF)],
        ],
    )
    def k(table_hbm, idx_hbm, out_hbm, idx_v, rows_v, semg):
        wid = lax.axis_index("s") * _NC + lax.axis_index("c")
        base = wid * per_w
        pltpu.sync_copy(idx_hbm.at[wid], idx_v)

        for b in range(NBUF):
            pltpu.async_copy(table_hbm.at[idx_v.at[b]], rows_v[b], semg[b])

        def body(i2, carry):
            for b in range(NBUF):
                i = i2 * NBUF + b
                pltpu.make_async_copy(
                    table_hbm.at[idx_v.at[i]], rows_v[b], semg[b]
                ).wait()
                pltpu.sync_copy(rows_v[b], out_hbm.at[pl.ds(base + i * _CH, _CH)])

                @pl.when(i + NBUF < n_ch)
                def _():
                    pltpu.async_copy(
                        table_hbm.at[idx_v.at[i + NBUF]], rows_v[b], semg[b]
                    )
            return carry

        lax.fori_loop(0, n_ch // NBUF, body, 0, unroll=False)

    return k(table, idx3)


def _triplet_stage(g, sbf, W_sbf, Wbig, bt):
    Tp, H = g.shape
    NSR = sbf.shape[1]
    NB = Wbig.shape[1] // H

    def body(g_r, sbf_r, wsbf_r, wbig_r, o_r):
        sbf2 = jnp.dot(sbf_r[...], wsbf_r[...], preferred_element_type=jnp.float32)
        big = jnp.dot(
            g_r[...].astype(jnp.bfloat16),
            wbig_r[...].astype(jnp.bfloat16),
            preferred_element_type=jnp.float32,
        )
        acc = jnp.zeros((bt, H), jnp.float32)
        for j in range(NB):
            acc = acc + big[:, j * H : (j + 1) * H] * sbf2[:, j : j + 1]
        o_r[...] = acc

    return pl.pallas_call(
        body,
        grid=(Tp // bt,),
        in_specs=[
            pl.BlockSpec((bt, H), lambda i: (i, 0)),
            pl.BlockSpec((bt, NSR), lambda i: (i, 0)),
            pl.BlockSpec((NSR, NB), lambda i: (0, 0)),
            pl.BlockSpec((H, NB * H), lambda i: (0, 0)),
        ],
        out_specs=pl.BlockSpec((bt, H), lambda i: (i, 0)),
        out_shape=jax.ShapeDtypeStruct((Tp, H), jnp.float32),
        compiler_params=pltpu.CompilerParams(
            dimension_semantics=("parallel",)
        ),
    )(g, sbf, W_sbf, Wbig)


def _sc_scatter(y, idx, E):
    Tp, H = y.shape
    F = 8
    P = H // F
    per_tile = Tp // _NS
    n_ch = per_tile // _CH
    e_per_tile = E // _NS
    ECH = 2000
    n_ech = e_per_tile // ECH
    half = P // _NC
    idx3 = idx.reshape(_NS, n_ch, _CH)
    zeros = jnp.zeros((ECH, F), jnp.float32)
    mesh = plsc.VectorSubcoreMesh(core_axis_name="c", subcore_axis_name="s")

    NBUF = 4

    @functools.partial(
        pl.kernel,
        out_type=jax.ShapeDtypeStruct((E, H), jnp.float32),
        mesh=mesh,
        scratch_types=[
            pltpu.VMEM((n_ch, _CH), jnp.int32),
            [pltpu.VMEM((_CH, F), jnp.float32) for _ in range(NBUF)],
            pltpu.VMEM((ECH, F), jnp.float32),
            pltpu.VMEM((ECH, F), jnp.float32),
            pltpu.VMEM_SHARED((E, F), jnp.float32),
            [pltpu.SemaphoreType.DMA for _ in range(NBUF)],
            pltpu.SemaphoreType.DMA,
        ],
        compiler_params=pltpu.CompilerParams(use_tc_tiling_on_sc=False),
    )
    def k(y_hbm, idx_hbm, z_hbm, out_hbm, idx_v, data_v, zbuf, fbuf, acc,
          semr, sem):
        cid = lax.axis_index("c")
        sid = lax.axis_index("s")
        pltpu.sync_copy(idx_hbm.at[sid], idx_v)
        pltpu.sync_copy(z_hbm, zbuf)
        row0 = sid * per_tile
        e0 = sid * e_per_tile

        def rd_src(col0, i):
            return y_hbm.at[pl.ds(row0 + i * _CH, _CH), pl.ds(col0, F)]

        for p in range(half):
            col0 = (cid * half + p) * F
            for q in range(n_ech):
                pltpu.sync_copy(zbuf, acc.at[pl.ds(e0 + q * ECH, ECH)])
            plsc.subcore_barrier()

            for b in range(NBUF):
                pltpu.async_copy(rd_src(col0, b), data_v[b], semr[b])

            def body(i2, carry):
                for b in range(NBUF):
                    i = i2 * NBUF + b
                    pltpu.make_async_copy(
                        rd_src(col0, i), data_v[b], semr[b]
                    ).wait()
                    pltpu.sync_copy(data_v[b], acc.at[idx_v.at[i]], add=True)

                    @pl.when(i + NBUF < n_ch)
                    def _():
                        pltpu.async_copy(
                            rd_src(col0, i + NBUF), data_v[b], semr[b]
                        )
                return carry

            lax.fori_loop(0, n_ch // NBUF, body, 0, unroll=False)
            plsc.subcore_barrier()
            for q in range(n_ech):
                pltpu.sync_copy(acc.at[pl.ds(e0 + q * ECH, ECH)], fbuf)
                pltpu.sync_copy(
                    fbuf,
                    out_hbm.at[pl.ds(e0 + q * ECH, ECH), pl.ds(col0, F)],
                )
            plsc.subcore_barrier()

    return k(y, idx3, zeros)


def _final_stage(x, scat, W_ji, b_ji, rb_w1, rb_b1, rb_w2, rb_b2,
                 W_lin, b_lin, ra1_w1, ra1_b1, ra1_w2, ra1_b2,
                 ra2_w1, ra2_b1, ra2_w2, ra2_b2, be):
    E, H = x.shape

    def body(x_r, sc_r, wji_r, bji_r, w1_r, B1_r, w2_r, B2_r,
             wl_r, bl_r, a1_r, ab1_r, a2_r, ab2_r, a3_r, ab3_r, a4_r, ab4_r,
             o_r):
        xv = x_r[...]
        h = _swish(jnp.dot(xv, wji_r[...], preferred_element_type=jnp.float32)
                   + bji_r[...]) + sc_r[...]
        t = _swish(jnp.dot(h, w1_r[...], preferred_element_type=jnp.float32)
                   + B1_r[...])
        h = h + _swish(jnp.dot(t, w2_r[...], preferred_element_type=jnp.float32)
                       + B2_r[...])
        h = _swish(jnp.dot(h, wl_r[...], preferred_element_type=jnp.float32)
                   + bl_r[...]) + xv
        t = _swish(jnp.dot(h, a1_r[...], preferred_element_type=jnp.float32)
                   + ab1_r[...])
        h = h + _swish(jnp.dot(t, a2_r[...], preferred_element_type=jnp.float32)
                       + ab2_r[...])
        t = _swish(jnp.dot(h, a3_r[...], preferred_element_type=jnp.float32)
                   + ab3_r[...])
        h = h + _swish(jnp.dot(t, a4_r[...], preferred_element_type=jnp.float32)
                       + ab4_r[...])
        o_r[...] = h

    wfull = pl.BlockSpec((H, H), lambda i: (0, 0))
    bfull = pl.BlockSpec((1, H), lambda i: (0, 0))
    return pl.pallas_call(
        body,
        grid=(E // be,),
        in_specs=[
            pl.BlockSpec((be, H), lambda i: (i, 0)),
            pl.BlockSpec((be, H), lambda i: (i, 0)),
            wfull, bfull, wfull, bfull, wfull, bfull,
            wfull, bfull, wfull, bfull, wfull, bfull,
            wfull, bfull, wfull, bfull,
        ],
        out_specs=pl.BlockSpec((be, H), lambda i: (i, 0)),
        out_shape=jax.ShapeDtypeStruct((E, H), jnp.float32),
        compiler_params=pltpu.CompilerParams(
            dimension_semantics=("parallel",)
        ),
    )(x, scat, W_ji, b_ji.reshape(1, H), rb_w1, rb_b1.reshape(1, H),
      rb_w2, rb_b2.reshape(1, H), W_lin, b_lin.reshape(1, H),
      ra1_w1, ra1_b1.reshape(1, H), ra1_w2, ra1_b2.reshape(1, H),
      ra2_w1, ra2_b1.reshape(1, H), ra2_w2, ra2_b2.reshape(1, H))


def kernel(x, rbf, sbf, idx_kj, idx_ji, W_rbf, W_sbf, W_kj, b_kj, W_ji, b_ji,
           W_bil, rb_w1, rb_b1, rb_w2, rb_b2, W_lin, b_lin,
           ra1_w1, ra1_b1, ra1_w2, ra1_b2, ra2_w1, ra2_b1, ra2_w2, ra2_b2):
    E, H = x.shape
    T = idx_kj.shape[0]
    ikj = idx_kj.astype(jnp.int32)
    iji = idx_ji.astype(jnp.int32)

    gran = _NW * _CH
    Tp = ((T + gran - 1) // gran) * gran
    pad = Tp - T
    if pad:
        fill = jnp.arange(pad, dtype=jnp.int32) % E
        ikj = jnp.concatenate([ikj, fill])
        iji = jnp.concatenate([iji, fill])
        sbf = jnp.pad(sbf, ((0, pad), (0, 0)))

    NB = W_bil.shape[1]
    Wbig = jnp.transpose(W_bil, (2, 1, 0)).reshape(H, NB * H)

    x_kj0 = _edge_stage(x, rbf, W_kj, b_kj, W_rbf, be=2000)
    g = _sc_gather(x_kj0, ikj)
    y = _triplet_stage(g, sbf, W_sbf, Wbig, bt=2048)
    scat = _sc_scatter(y, iji, E)
    return _final_stage(x, scat, W_ji, b_ji, rb_w1, rb_b1, rb_w2, rb_b2,
                        W_lin, b_lin, ra1_w1, ra1_b1, ra1_w2, ra1_b2,
                        ra2_w1, ra2_b1, ra2_w2, ra2_b2, be=2000)
